# baseline (speedup 1.0000x reference)
import math
import numpy as np
import jax
import jax.numpy as jnp
from jax import lax
from jax.experimental import pallas as pl
from jax.experimental.pallas import tpu as pltpu
from jax.experimental.pallas import tpu_sc as plsc

H = 128
W = 128
THETA_RES = 2.0
RHO_RES = 2.0
NUMANGLE = int(math.ceil(180.0 / THETA_RES))
_D = math.sqrt((H - 1) ** 2 + (W - 1) ** 2)
NUMRHO = int(math.ceil(2.0 * _D / RHO_RES) + 1)
NRPAD = 184

HW = H * W
NC = 256

SC_A = 8
TC_A = NUMANGLE - SC_A

NLANE = 16
NTILE = 32
CPT = NC // NTILE
P = 512
NCHUNK = HW // P
ACOL = SC_A * NRPAD
ACOLP = 1536
ACC_WORDS = CPT * ACOLP

KC = 2048
NKC = HW // KC
NRW = 128


def _rho_index_table() -> np.ndarray:
    thetas = np.arange(NUMANGLE, dtype=np.float64) * (np.pi / NUMANGLE)
    cos_t = np.cos(thetas)
    sin_t = np.sin(thetas)
    ys, xs = np.meshgrid(np.arange(H), np.arange(W), indexing="ij")
    xc = (xs - W // 2).reshape(-1).astype(np.float64)
    yc = (ys - H // 2).reshape(-1).astype(np.float64)
    rho = cos_t[:, None] * xc[None, :] + sin_t[:, None] * yc[None, :]
    ridx = np.round(rho / RHO_RES).astype(np.int32) + NUMRHO // 2
    return np.clip(ridx, 0, NUMRHO - 1)


_RIDX = _rho_index_table()
_RIDXT_SC = np.tile(
    np.ascontiguousarray(_RIDX[TC_A:, :].T), (1, 2)).reshape(-1)
_C_A = np.clip(_RIDX[:TC_A].min(axis=1), 0, NUMRHO - NRW)
_RIDX_TC = np.ascontiguousarray(
    (_RIDX[:TC_A] - _C_A[:, None])).reshape(TC_A, NKC, 1, KC)
_EXPAND = np.zeros((TC_A, NRPAD, NRW), np.float32)
for _a in range(TC_A):
    _EXPAND[_a, _C_A[_a]:_C_A[_a] + NRW, :] = np.eye(NRW, dtype=np.float32)


def _dht_sc_body(data_hbm, ridxt_hbm, out_hbm,
                 idx_v0, idx_v1, dat_v0, dat_v1, acc_v, sem0, sem1):
    cid = lax.axis_index("c")
    sid = lax.axis_index("s")
    wid = sid * 2 + cid
    c0 = wid * CPT
    lane = lax.broadcasted_iota(jnp.int32, (NLANE,), 0)
    half = lane // SC_A
    base = lax.rem(lane, SC_A) * NRPAD + half * ACOLP
    coff = [base + jnp.full((NLANE,), 2 * cp * ACOLP, jnp.int32)
            for cp in range(CPT // 2)]
    dsel = [half * P + jnp.full((NLANE,), 2 * cp * P, jnp.int32)
            for cp in range(CPT // 2)]
    idx_b = (idx_v0, idx_v1)
    dat_b = (dat_v0, dat_v1)
    sem_b = (sem0, sem1)

    def start_chunk(t, b):
        k = lax.rem(wid + t, NCHUNK)
        pix0 = k * P
        pltpu.async_copy(ridxt_hbm.at[pl.ds(pix0 * NLANE, P * NLANE)],
                         idx_b[b], sem_b[b])
        for c in range(CPT):
            pltpu.async_copy(data_hbm.at[pl.ds((c0 + c) * HW + pix0, P)],
                             dat_b[b].at[pl.ds(c * P, P)], sem_b[b])

    def wait_chunk(b):
        pltpu.make_async_copy(ridxt_hbm.at[pl.ds(0, P * NLANE)],
                              idx_b[b], sem_b[b]).wait()
        for c in range(CPT):
            pltpu.make_async_copy(data_hbm.at[pl.ds(0, P)],
                                  dat_b[b].at[pl.ds(c * P, P)],
                                  sem_b[b]).wait()

    def compute_chunk(b):
        idx_v = idx_b[b]
        dat_v = dat_b[b]

        def pix_body(i, c2):
            r16 = idx_v[pl.ds(i * NLANE, NLANE)]
            ivec = jnp.full((NLANE,), i, jnp.int32)
            for cp in range(CPT // 2):
                xv = plsc.load_gather(dat_v, [ivec + dsel[cp]])
                plsc.addupdate_scatter(acc_v, [r16 + coff[cp]], xv)
            return c2
        lax.fori_loop(0, P, pix_body, 0)

    def zero_body(j, carry):
        acc_v[pl.ds(j * NLANE, NLANE)] = jnp.zeros((NLANE,), jnp.float32)
        return carry
    lax.fori_loop(0, ACC_WORDS // NLANE, zero_body, 0)

    start_chunk(0, 0)
    start_chunk(1, 1)

    def chunk_body(tt, carry):
        for b in range(2):
            wait_chunk(b)
            compute_chunk(b)
            start_chunk(tt * 2 + b + 2, b)
        return carry
    lax.fori_loop(0, NCHUNK // 2, chunk_body, 0)
    wait_chunk(0)
    wait_chunk(1)

    for c in range(CPT):
        pltpu.sync_copy(acc_v.at[pl.ds(c * ACOLP, ACOLP)],
                        out_hbm.at[pl.ds((c0 + c) * ACOLP, ACOLP)])


def _dht_sc(data, ridxt):
    mesh = plsc.VectorSubcoreMesh(core_axis_name="c", subcore_axis_name="s")
    return pl.kernel(
        _dht_sc_body,
        mesh=mesh,
        out_type=jax.ShapeDtypeStruct((NC * ACOLP,), jnp.float32),
        compiler_params=pltpu.CompilerParams(needs_layout_passes=False),
        cost_estimate=pl.CostEstimate(
            flops=400_000_000, transcendentals=0, bytes_accessed=50_000_000),
        scratch_types=[
            pltpu.VMEM((P * NLANE,), jnp.int32),
            pltpu.VMEM((P * NLANE,), jnp.int32),
            pltpu.VMEM((CPT * P,), jnp.float32),
            pltpu.VMEM((CPT * P,), jnp.float32),
            pltpu.VMEM((ACC_WORDS,), jnp.float32),
            pltpu.SemaphoreType.DMA,
            pltpu.SemaphoreType.DMA,
        ],
    )(data, ridxt)


def _dht_tc_body(ridx_ref, exp_ref, data_ref, out_ref):
    rho_iota = lax.broadcasted_iota(jnp.int32, (NRW, KC), 0)
    acc = jnp.zeros((NRW, NC), jnp.float32)
    for k in range(NKC):
        r = ridx_ref[0, k, 0, :]
        onehot = (r[None, :] == rho_iota).astype(jnp.bfloat16)
        acc += jnp.dot(onehot, data_ref[pl.ds(k * KC, KC), :],
                       preferred_element_type=jnp.float32)
    out_ref[0, :, :] = jnp.dot(exp_ref[0], acc,
                               preferred_element_type=jnp.float32)


def _dht_tc(data16, ridx_tc):
    return pl.pallas_call(
        _dht_tc_body,
        grid=(TC_A,),
        in_specs=[
            pl.BlockSpec((1, NKC, 1, KC), lambda a: (a, 0, 0, 0)),
            pl.BlockSpec((1, NRPAD, NRW), lambda a: (a, 0, 0)),
            pl.BlockSpec((HW, NC), lambda a: (0, 0)),
        ],
        out_specs=pl.BlockSpec((1, NRPAD, NC), lambda a: (a, 0, 0)),
        out_shape=jax.ShapeDtypeStruct((TC_A, NRPAD, NC), jnp.float32),
        compiler_params=pltpu.CompilerParams(
            dimension_semantics=("arbitrary",)),
    )(ridx_tc, jnp.asarray(_EXPAND), data16)


def kernel(feat):
    N, C, Hh, Ww = feat.shape
    data = feat.reshape(NC, HW)
    out_sc = _dht_sc(data.reshape(-1), jnp.asarray(_RIDXT_SC))
    out_tc = _dht_tc(data.T.astype(jnp.bfloat16), jnp.asarray(_RIDX_TC))
    sc = out_sc.reshape(NC, ACOLP)[:, :ACOL]
    sc = sc.reshape(NC, SC_A, NRPAD)[:, :, :NUMRHO]
    tc = jnp.transpose(out_tc[:, :NUMRHO, :], (2, 0, 1))
    out = jnp.concatenate([tc, sc], axis=1)
    return out.reshape(N, C, NUMANGLE, NUMRHO)


# speedup vs baseline: 1.2813x; 1.2813x over previous
import math
import numpy as np
import jax
import jax.numpy as jnp
from jax import lax
from jax.experimental import pallas as pl
from jax.experimental.pallas import tpu as pltpu
from jax.experimental.pallas import tpu_sc as plsc

H = 128
W = 128
THETA_RES = 2.0
RHO_RES = 2.0
NUMANGLE = int(math.ceil(180.0 / THETA_RES))
_D = math.sqrt((H - 1) ** 2 + (W - 1) ** 2)
NUMRHO = int(math.ceil(2.0 * _D / RHO_RES) + 1)
NRPAD = 184

HW = H * W
NC = 256

SC_A = 8
TC_A = NUMANGLE - SC_A

NLANE = 16
NTILE = 32
CPT = NC // NTILE
P = 512
NCHUNK = HW // P
ACOL = SC_A * NRPAD
ACOLP = 1536
ACC_WORDS = CPT * ACOLP

KC = 2048
NKC = HW // KC
NRW = 128


def _rho_index_table() -> np.ndarray:
    thetas = np.arange(NUMANGLE, dtype=np.float64) * (np.pi / NUMANGLE)
    cos_t = np.cos(thetas)
    sin_t = np.sin(thetas)
    ys, xs = np.meshgrid(np.arange(H), np.arange(W), indexing="ij")
    xc = (xs - W // 2).reshape(-1).astype(np.float64)
    yc = (ys - H // 2).reshape(-1).astype(np.float64)
    rho = cos_t[:, None] * xc[None, :] + sin_t[:, None] * yc[None, :]
    ridx = np.round(rho / RHO_RES).astype(np.int32) + NUMRHO // 2
    return np.clip(ridx, 0, NUMRHO - 1)


_RIDX = _rho_index_table()
_RIDXT_SC = np.tile(
    np.ascontiguousarray(_RIDX[TC_A:, :].T), (1, 2)).reshape(-1)
_C_A = np.clip(_RIDX[:TC_A].min(axis=1), 0, NUMRHO - NRW)
_RIDX_TC = np.ascontiguousarray(
    (_RIDX[:TC_A] - _C_A[:, None])).reshape(TC_A, NKC, 1, KC)
_EXPAND = np.zeros((TC_A, NRPAD, NRW), np.float32)
for _a in range(TC_A):
    _EXPAND[_a, _C_A[_a]:_C_A[_a] + NRW, :] = np.eye(NRW, dtype=np.float32)


def _dht_sc_body(data_hbm, ridxt_hbm, out_hbm,
                 idx_v0, idx_v1, dat_v0, dat_v1, acc_v, sem0, sem1):
    cid = lax.axis_index("c")
    sid = lax.axis_index("s")
    wid = sid * 2 + cid
    c0 = wid * CPT
    lane = lax.broadcasted_iota(jnp.int32, (NLANE,), 0)
    half = lane // SC_A
    base = lax.rem(lane, SC_A) * NRPAD + half * ACOLP
    coff = [base + jnp.full((NLANE,), 2 * cp * ACOLP, jnp.int32)
            for cp in range(CPT // 2)]
    dsel = [half * P + jnp.full((NLANE,), 2 * cp * P, jnp.int32)
            for cp in range(CPT // 2)]
    idx_b = (idx_v0, idx_v1)
    dat_b = (dat_v0, dat_v1)
    sem_b = (sem0, sem1)

    def start_chunk(t, b):
        k = lax.rem(wid + t, NCHUNK)
        pix0 = k * P
        pltpu.async_copy(ridxt_hbm.at[pl.ds(pix0 * NLANE, P * NLANE)],
                         idx_b[b], sem_b[b])
        for c in range(CPT):
            pltpu.async_copy(data_hbm.at[pl.ds((c0 + c) * HW + pix0, P)],
                             dat_b[b].at[pl.ds(c * P, P)], sem_b[b])

    def wait_chunk(b):
        pltpu.make_async_copy(ridxt_hbm.at[pl.ds(0, P * NLANE)],
                              idx_b[b], sem_b[b]).wait()
        for c in range(CPT):
            pltpu.make_async_copy(data_hbm.at[pl.ds(0, P)],
                                  dat_b[b].at[pl.ds(c * P, P)],
                                  sem_b[b]).wait()

    def compute_chunk(b):
        idx_v = idx_b[b]
        dat_v = dat_b[b]

        def loads(i):
            r16 = idx_v[pl.ds(i * NLANE, NLANE)]
            ivec = jnp.full((NLANE,), i, jnp.int32)
            xs = [plsc.load_gather(dat_v, [ivec + dsel[cp]])
                  for cp in range(CPT // 2)]
            return r16, xs

        def pix_body(ib, c2):
            i0 = ib * NLANE
            cur = loads(i0)
            for l in range(NLANE):
                r16, xs = cur
                if l + 1 < NLANE:
                    cur = loads(i0 + l + 1)
                for cp in range(CPT // 2):
                    plsc.addupdate_scatter(acc_v, [r16 + coff[cp]], xs[cp])
            return c2
        lax.fori_loop(0, P // NLANE, pix_body, 0)

    def zero_body(j, carry):
        acc_v[pl.ds(j * NLANE, NLANE)] = jnp.zeros((NLANE,), jnp.float32)
        return carry
    lax.fori_loop(0, ACC_WORDS // NLANE, zero_body, 0)

    start_chunk(0, 0)
    start_chunk(1, 1)

    def chunk_body(tt, carry):
        for b in range(2):
            wait_chunk(b)
            compute_chunk(b)
            start_chunk(tt * 2 + b + 2, b)
        return carry
    lax.fori_loop(0, NCHUNK // 2, chunk_body, 0)
    wait_chunk(0)
    wait_chunk(1)

    for c in range(CPT):
        pltpu.sync_copy(acc_v.at[pl.ds(c * ACOLP, ACOLP)],
                        out_hbm.at[pl.ds((c0 + c) * ACOLP, ACOLP)])


def _dht_sc(data, ridxt):
    mesh = plsc.VectorSubcoreMesh(core_axis_name="c", subcore_axis_name="s")
    return pl.kernel(
        _dht_sc_body,
        mesh=mesh,
        out_type=jax.ShapeDtypeStruct((NC * ACOLP,), jnp.float32),
        compiler_params=pltpu.CompilerParams(needs_layout_passes=False),
        cost_estimate=pl.CostEstimate(
            flops=400_000_000, transcendentals=0, bytes_accessed=50_000_000),
        scratch_types=[
            pltpu.VMEM((P * NLANE,), jnp.int32),
            pltpu.VMEM((P * NLANE,), jnp.int32),
            pltpu.VMEM((CPT * P,), jnp.float32),
            pltpu.VMEM((CPT * P,), jnp.float32),
            pltpu.VMEM((ACC_WORDS,), jnp.float32),
            pltpu.SemaphoreType.DMA,
            pltpu.SemaphoreType.DMA,
        ],
    )(data, ridxt)


def _dht_tc_body(ridx_ref, exp_ref, data_ref, out_ref):
    rho_iota = lax.broadcasted_iota(jnp.int32, (NRW, KC), 0)
    acc = jnp.zeros((NRW, NC), jnp.float32)
    for k in range(NKC):
        r = ridx_ref[0, k, 0, :]
        onehot = (r[None, :] == rho_iota).astype(jnp.bfloat16)
        acc += jnp.dot(onehot, data_ref[pl.ds(k * KC, KC), :],
                       preferred_element_type=jnp.float32)
    out_ref[0, :, :] = jnp.dot(exp_ref[0], acc,
                               preferred_element_type=jnp.float32)


def _dht_tc(data16, ridx_tc):
    return pl.pallas_call(
        _dht_tc_body,
        grid=(TC_A,),
        in_specs=[
            pl.BlockSpec((1, NKC, 1, KC), lambda a: (a, 0, 0, 0)),
            pl.BlockSpec((1, NRPAD, NRW), lambda a: (a, 0, 0)),
            pl.BlockSpec((HW, NC), lambda a: (0, 0)),
        ],
        out_specs=pl.BlockSpec((1, NRPAD, NC), lambda a: (a, 0, 0)),
        out_shape=jax.ShapeDtypeStruct((TC_A, NRPAD, NC), jnp.float32),
        compiler_params=pltpu.CompilerParams(
            dimension_semantics=("arbitrary",)),
    )(ridx_tc, jnp.asarray(_EXPAND), data16)


def kernel(feat):
    N, C, Hh, Ww = feat.shape
    data = feat.reshape(NC, HW)
    out_sc = _dht_sc(data.reshape(-1), jnp.asarray(_RIDXT_SC))
    out_tc = _dht_tc(data.T.astype(jnp.bfloat16), jnp.asarray(_RIDX_TC))
    sc = out_sc.reshape(NC, ACOLP)[:, :ACOL]
    sc = sc.reshape(NC, SC_A, NRPAD)[:, :, :NUMRHO]
    tc = jnp.transpose(out_tc[:, :NUMRHO, :], (2, 0, 1))
    out = jnp.concatenate([tc, sc], axis=1)
    return out.reshape(N, C, NUMANGLE, NUMRHO)


# speedup vs baseline: 3.5117x; 2.7407x over previous
import math
import numpy as np
import jax
import jax.numpy as jnp
from jax import lax
from jax.experimental import pallas as pl
from jax.experimental.pallas import tpu as pltpu
from jax.experimental.pallas import tpu_sc as plsc

H = 128
W = 128
THETA_RES = 2.0
RHO_RES = 2.0
NUMANGLE = int(math.ceil(180.0 / THETA_RES))
_D = math.sqrt((H - 1) ** 2 + (W - 1) ** 2)
NUMRHO = int(math.ceil(2.0 * _D / RHO_RES) + 1)
NRPAD = 184

HW = H * W
NC = 256

SC_A = 16
TC_A = NUMANGLE - SC_A

NLANE = 16
NGRP = SC_A // NLANE
NTILE = 32
CPT = NC // NTILE
P = 512
NCHUNK = HW // P
ACOL = SC_A * NRPAD
ACC_WORDS = CPT * ACOL

KC = 2048
NKC = HW // KC
NRW = 128


def _rho_index_table() -> np.ndarray:
    thetas = np.arange(NUMANGLE, dtype=np.float64) * (np.pi / NUMANGLE)
    cos_t = np.cos(thetas)
    sin_t = np.sin(thetas)
    ys, xs = np.meshgrid(np.arange(H), np.arange(W), indexing="ij")
    xc = (xs - W // 2).reshape(-1).astype(np.float64)
    yc = (ys - H // 2).reshape(-1).astype(np.float64)
    rho = cos_t[:, None] * xc[None, :] + sin_t[:, None] * yc[None, :]
    ridx = np.round(rho / RHO_RES).astype(np.int32) + NUMRHO // 2
    return np.clip(ridx, 0, NUMRHO - 1)


_RIDX = _rho_index_table()
_RIDXT_SC = np.ascontiguousarray(_RIDX[TC_A:, :].T).reshape(-1)
_C_A = np.clip(_RIDX[:TC_A].min(axis=1), 0, NUMRHO - NRW)
_RIDX_TC = np.ascontiguousarray(
    (_RIDX[:TC_A] - _C_A[:, None])).reshape(TC_A, NKC, 1, KC)
_EXPAND = np.zeros((TC_A, NRPAD, NRW), np.float32)
for _a in range(TC_A):
    _EXPAND[_a, _C_A[_a]:_C_A[_a] + NRW, :] = np.eye(NRW, dtype=np.float32)


def _dht_sc_body(data_hbm, ridxt_hbm, out_hbm,
                 idx_v0, idx_v1, dat_v0, dat_v1, acc_v, sem0, sem1):
    cid = lax.axis_index("c")
    sid = lax.axis_index("s")
    wid = sid * 2 + cid
    c0 = wid * CPT
    lane = lax.broadcasted_iota(jnp.int32, (NLANE,), 0)
    base_g = [(g * NLANE + lane) * NRPAD for g in range(NGRP)]
    coff = [jnp.full((NLANE,), c * ACOL, jnp.int32) for c in range(CPT)]
    idx_b = (idx_v0, idx_v1)
    dat_b = (dat_v0, dat_v1)
    sem_b = (sem0, sem1)

    def start_chunk(t, b):
        k = lax.rem(wid + t, NCHUNK)
        pix0 = k * P
        pltpu.async_copy(ridxt_hbm.at[pl.ds(pix0 * SC_A, P * SC_A)],
                         idx_b[b], sem_b[b])
        for c in range(CPT):
            pltpu.async_copy(data_hbm.at[pl.ds((c0 + c) * HW + pix0, P)],
                             dat_b[b].at[pl.ds(c * P, P)], sem_b[b])

    def wait_chunk(b):
        pltpu.make_async_copy(ridxt_hbm.at[pl.ds(0, P * SC_A)],
                              idx_b[b], sem_b[b]).wait()
        for c in range(CPT):
            pltpu.make_async_copy(data_hbm.at[pl.ds(0, P)],
                                  dat_b[b].at[pl.ds(c * P, P)],
                                  sem_b[b]).wait()

    def compute_chunk(b):
        idx_v = idx_b[b]
        dat_v = dat_b[b]

        def pix_body(ib, c2):
            i0 = ib * NLANE
            dvs = [dat_v[pl.ds(c * P + i0, NLANE)] for c in range(CPT)]
            rs = [idx_v[pl.ds(i0 * SC_A + g * NLANE, NLANE)]
                  for g in range(NGRP)]
            for l in range(NLANE):
                rs_cur = rs
                if l + 1 < NLANE:
                    rs = [idx_v[pl.ds((i0 + l + 1) * SC_A + g * NLANE,
                                      NLANE)] for g in range(NGRP)]
                xs = [jnp.full((NLANE,), dvs[c][l], jnp.float32)
                      for c in range(CPT)]
                for g in range(NGRP):
                    addr = base_g[g] + rs_cur[g]
                    for c in range(CPT):
                        plsc.addupdate_scatter(acc_v, [addr + coff[c]], xs[c])
            return c2
        lax.fori_loop(0, P // NLANE, pix_body, 0)

    def zero_body(j, carry):
        acc_v[pl.ds(j * NLANE, NLANE)] = jnp.zeros((NLANE,), jnp.float32)
        return carry
    lax.fori_loop(0, ACC_WORDS // NLANE, zero_body, 0)

    start_chunk(0, 0)
    start_chunk(1, 1)

    def chunk_body(tt, carry):
        for b in range(2):
            wait_chunk(b)
            compute_chunk(b)
            start_chunk(tt * 2 + b + 2, b)
        return carry
    lax.fori_loop(0, NCHUNK // 2, chunk_body, 0)
    wait_chunk(0)
    wait_chunk(1)

    for c in range(CPT):
        pltpu.sync_copy(acc_v.at[pl.ds(c * ACOL, ACOL)],
                        out_hbm.at[pl.ds((c0 + c) * ACOL, ACOL)])


def _dht_sc(data, ridxt):
    mesh = plsc.VectorSubcoreMesh(core_axis_name="c", subcore_axis_name="s")
    return pl.kernel(
        _dht_sc_body,
        mesh=mesh,
        out_type=jax.ShapeDtypeStruct((NC * ACOL,), jnp.float32),
        compiler_params=pltpu.CompilerParams(needs_layout_passes=False),
        cost_estimate=pl.CostEstimate(
            flops=400_000_000, transcendentals=0, bytes_accessed=50_000_000),
        scratch_types=[
            pltpu.VMEM((P * SC_A,), jnp.int32),
            pltpu.VMEM((P * SC_A,), jnp.int32),
            pltpu.VMEM((CPT * P,), jnp.float32),
            pltpu.VMEM((CPT * P,), jnp.float32),
            pltpu.VMEM((ACC_WORDS,), jnp.float32),
            pltpu.SemaphoreType.DMA,
            pltpu.SemaphoreType.DMA,
        ],
    )(data, ridxt)


def _dht_tc_body(ridx_ref, exp_ref, data_ref, out_ref):
    rho_iota = lax.broadcasted_iota(jnp.int32, (NRW, KC), 0)
    acc = jnp.zeros((NRW, NC), jnp.float32)
    for k in range(NKC):
        r = ridx_ref[0, k, 0, :]
        onehot = (r[None, :] == rho_iota).astype(jnp.bfloat16)
        acc += jnp.dot(onehot, data_ref[pl.ds(k * KC, KC), :],
                       preferred_element_type=jnp.float32)
    out_ref[0, :, :] = jnp.dot(exp_ref[0], acc,
                               preferred_element_type=jnp.float32)


def _dht_tc(data16, ridx_tc):
    return pl.pallas_call(
        _dht_tc_body,
        grid=(TC_A,),
        in_specs=[
            pl.BlockSpec((1, NKC, 1, KC), lambda a: (a, 0, 0, 0)),
            pl.BlockSpec((1, NRPAD, NRW), lambda a: (a, 0, 0)),
            pl.BlockSpec((HW, NC), lambda a: (0, 0)),
        ],
        out_specs=pl.BlockSpec((1, NRPAD, NC), lambda a: (a, 0, 0)),
        out_shape=jax.ShapeDtypeStruct((TC_A, NRPAD, NC), jnp.float32),
        compiler_params=pltpu.CompilerParams(
            dimension_semantics=("arbitrary",)),
    )(ridx_tc, jnp.asarray(_EXPAND), data16)


def kernel(feat):
    N, C, Hh, Ww = feat.shape
    data = feat.reshape(NC, HW)
    out_sc = _dht_sc(data.reshape(-1), jnp.asarray(_RIDXT_SC))
    out_tc = _dht_tc(data.T.astype(jnp.bfloat16), jnp.asarray(_RIDX_TC))
    sc = out_sc.reshape(NC, SC_A, NRPAD)[:, :, :NUMRHO]
    tc = jnp.transpose(out_tc[:, :NUMRHO, :], (2, 0, 1))
    out = jnp.concatenate([tc, sc], axis=1)
    return out.reshape(N, C, NUMANGLE, NUMRHO)
